# baseline (speedup 1.0000x reference)
import functools

import jax
import jax.numpy as jnp
from jax import lax
from jax.experimental import pallas as pl
from jax.experimental.pallas import tpu as pltpu
from jax.experimental.pallas import tpu_sc as plsc

_NC = 2
_NS = 16
_NW = _NC * _NS
_L = 16


def _pad_cols(x, c):
    if x.shape[-1] == c:
        return x
    pad = [(0, 0)] * (x.ndim - 1) + [(0, c - x.shape[-1])]
    return jnp.pad(x, pad)


def _dist_body(q_ref, rt_ref, out_ref):
    q = q_ref[0]
    rt = rt_ref[0]
    dot = lax.dot_general(q, rt, (((1,), (0,)), ((), ())),
                          preferred_element_type=jnp.float32)
    qn = jnp.sum(q * q, axis=1, keepdims=True)
    rn = jnp.sum(rt * rt, axis=0, keepdims=True)
    out_ref[0] = qn + rn - 2.0 * dot


def _dist_pallas(queries, refs):
    G, P, _ = queries.shape
    N = refs.shape[1]
    refsT = jnp.swapaxes(refs, 1, 2)
    TQ = min(P, max(128, (1 << 23) // (4 * N)))
    grid = (G, P // TQ)
    return pl.pallas_call(
        _dist_body,
        grid=grid,
        in_specs=[
            pl.BlockSpec((1, TQ, 3), lambda g, t: (g, t, 0)),
            pl.BlockSpec((1, 3, N), lambda g, t: (g, 0, 0)),
        ],
        out_specs=pl.BlockSpec((1, TQ, N), lambda g, t: (g, t, 0)),
        out_shape=jax.ShapeDtypeStruct((G, P, N), jnp.float32),
    )(queries, refsT)


def _merge16(ad, ai, cd, ci):
    rd = lax.rev(cd, (0,))
    ri = lax.rev(ci, (0,))
    m = ad <= rd
    lo_d = jnp.where(m, ad, rd)
    lo_i = jnp.where(m, ai, ri)
    hi_d = jnp.where(m, rd, ad)
    hi_i = jnp.where(m, ri, ai)
    return lo_d, lo_i, hi_d, hi_i


def _topk_gather(D, k, npoints, nrefs, table):
    R, N = D.shape
    rpw = R // _NW
    npairs = rpw // 2
    CH = 8
    nchunk = N // (CH * _L)
    log2np = npoints.bit_length() - 1
    log2n = nrefs.bit_length() - 1
    nh = k // _L
    CAP = 1024
    C = table.shape[1]
    mesh = plsc.VectorSubcoreMesh(core_axis_name="c", subcore_axis_name="s")

    @functools.partial(
        pl.kernel,
        out_type=jax.ShapeDtypeStruct((R * k, C), jnp.float32),
        mesh=mesh,
        compiler_params=pltpu.CompilerParams(needs_layout_passes=False,
                                             use_tc_tiling_on_sc=False),
        scratch_types=[
            pltpu.VMEM((2, N), jnp.float32),
            pltpu.VMEM((CAP + 2 * _L,), jnp.float32),
            pltpu.VMEM((CAP + 2 * _L,), jnp.int32),
            pltpu.VMEM((k,), jnp.float32),
            pltpu.VMEM((2, k), jnp.int32),
            pltpu.VMEM((2, k, C), jnp.float32),
            pltpu.SemaphoreType.DMA,
            pltpu.SemaphoreType.DMA,
            pltpu.SemaphoreType.DMA,
            pltpu.SemaphoreType.DMA,
        ],
    )
    def kern(d_hbm, t_hbm, out_hbm, rowbuf, candd, candi, topd, topi2,
             grows, sem0, sem1, gsem0, gsem1):
        wid = lax.axis_index("s") * _NC + lax.axis_index("c")
        base = wid * rpw
        big = jnp.full((_L,), 3.0e38, jnp.float32)
        zero = jnp.zeros((_L,), jnp.int32)
        ones = jnp.full((_L,), 1, jnp.int32)
        capv = jnp.full((_L,), CAP - 1, jnp.int32)

        def merge_cand(topi, cd0, ci0):
            cd, ci = plsc.sort_key_val(cd0, ci0)
            if nh == 1:
                ad = topd[pl.ds(0, _L)]
                ai = topi[pl.ds(0, _L)]
                lo_d, lo_i, _hd, _hi = _merge16(ad, ai, cd, ci)
                nd, ni = plsc.sort_key_val(lo_d, lo_i)
                topd[pl.ds(0, _L)] = nd
                topi[pl.ds(0, _L)] = ni
            else:
                a0d = topd[pl.ds(0, _L)]
                a0i = topi[pl.ds(0, _L)]
                a1d = topd[pl.ds(_L, _L)]
                a1i = topi[pl.ds(_L, _L)]
                ld, li, _hd, _hi = _merge16(a1d, a1i, cd, ci)
                sd, si = plsc.sort_key_val(ld, li)
                l0d, l0i, h0d, h0i = _merge16(a0d, a0i, sd, si)
                n0d, n0i = plsc.sort_key_val(l0d, l0i)
                n1d, n1i = plsc.sort_key_val(h0d, h0i)
                topd[pl.ds(0, _L)] = n0d
                topi[pl.ds(0, _L)] = n0i
                topd[pl.ds(_L, _L)] = n1d
                topi[pl.ds(_L, _L)] = n1i

        def process(phase, row):
            topi = topi2.at[phase]
            for h in range(nh):
                topd[pl.ds(h * _L, _L)] = big
                topi[pl.ds(h * _L, _L)] = zero

            def folda(c, carry):
                a1, a2 = carry
                b = c * (CH * _L)
                for t in range(CH):
                    v = rowbuf[phase, pl.ds(b + t * _L, _L)]
                    a2 = jnp.minimum(a2, jnp.maximum(a1, v))
                    a1 = jnp.minimum(a1, v)
                return a1, a2

            a1, a2 = lax.fori_loop(0, nchunk, folda, (big, big))
            s2, _si = plsc.sort_key_val(a2, zero)
            tvec = jnp.broadcast_to(s2[_L - 1], (_L,))

            def collect(c, curv):
                b = c * (CH * _L)
                ss = []
                for t in range(CH):
                    v = rowbuf[phase, pl.ds(b + t * _L, _L)]
                    m = v <= tvec
                    ss.append((v, m, plsc.cumsum(jnp.where(m, ones, zero))))
                cur = curv
                for t in range(CH):
                    v, m, s = ss[t]
                    idx = jnp.minimum(cur + s - 1, capv)
                    vi = b + t * _L + lax.iota(jnp.int32, _L)
                    plsc.store_scatter(candd, [idx], v, mask=m)
                    plsc.store_scatter(candi, [idx], vi, mask=m)
                    cur = cur + jnp.broadcast_to(s[_L - 1], (_L,))
                return cur

            curv = lax.fori_loop(0, nchunk, collect, zero)

            mtot = jnp.minimum(jnp.minimum(curv, capv)[0], jnp.int32(CAP))
            candd[pl.ds(mtot, _L)] = big
            nb = lax.shift_right_logical(mtot + (_L - 1), 4)

            def mergeb(i, _):
                b = i * _L
                merge_cand(topi, candd[pl.ds(b, _L)], candi[pl.ds(b, _L)])
                return 0

            lax.fori_loop(0, nb, mergeb, 0)

            off = lax.shift_left(lax.shift_right_logical(row, log2np), log2n)
            for h in range(nh):
                topi[pl.ds(h * _L, _L)] = topi[pl.ds(h * _L, _L)] + off

        pltpu.async_copy(d_hbm.at[base], rowbuf.at[0], sem0)

        def pair(p, _):
            row0 = base + 2 * p
            pltpu.make_async_copy(d_hbm.at[row0], rowbuf.at[0], sem0).wait()
            pltpu.async_copy(d_hbm.at[row0 + 1], rowbuf.at[1], sem1)
            process(0, row0)
            g0 = pltpu.async_copy(t_hbm.at[topi2.at[0]], grows.at[0], gsem0)
            pltpu.make_async_copy(d_hbm.at[row0 + 1], rowbuf.at[1],
                                  sem1).wait()

            @pl.when(p + 1 < npairs)
            def _():
                pltpu.async_copy(d_hbm.at[row0 + 2], rowbuf.at[0], sem0)

            process(1, row0 + 1)
            g1 = pltpu.async_copy(t_hbm.at[topi2.at[1]], grows.at[1], gsem1)
            g0.wait()
            pltpu.sync_copy(grows.at[0], out_hbm.at[pl.ds(row0 * k, k)])
            g1.wait()
            pltpu.sync_copy(grows.at[1], out_hbm.at[pl.ds((row0 + 1) * k, k)])
            return 0

        lax.fori_loop(0, npairs, pair, 0)

    return kern(D, table)


def _sa_body(x_ref, q_ref, w1x, w1f, b1, w2, b2, w3, b3, out_ref, *, T, k):
    x = x_ref[...]
    q = q_ref[...]
    h1 = w1x.shape[1]
    qb = jnp.broadcast_to(q[:, None, :], (T, k, 3)).reshape(T * k, 3)
    xr = x[:, :3] - qb
    h = jnp.maximum(xr @ w1x[...] + x @ w1f[...] + b1[...], 0.0)
    h = jnp.maximum(h @ w2[...] + b2[...], 0.0)
    h = jnp.maximum(h @ w3[...] + b3[...], 0.0)
    out_ref[...] = jnp.max(h.reshape(T, k, h.shape[-1]), axis=1)


def _sa_mlp_pallas(gath, q, layers, k, Cp):
    (W1, b1), (W2, b2), (W3, b3) = layers
    R = q.shape[0]
    T = 64 if k == 32 else 128
    H1, H2, H3 = W1.shape[1], W2.shape[1], W3.shape[1]
    W1x = W1[:3]
    W1f = _pad_cols(W1[3:].T, 0).T if False else jnp.concatenate(
        [jnp.zeros((3, H1), jnp.float32), W1[3:],
         jnp.zeros((Cp - W1.shape[0], H1), jnp.float32)], axis=0)
    grid = (R // T,)
    return pl.pallas_call(
        functools.partial(_sa_body, T=T, k=k),
        grid=grid,
        in_specs=[
            pl.BlockSpec((T * k, Cp), lambda t: (t, 0)),
            pl.BlockSpec((T, 3), lambda t: (t, 0)),
            pl.BlockSpec(W1x.shape, lambda t: (0, 0)),
            pl.BlockSpec(W1f.shape, lambda t: (0, 0)),
            pl.BlockSpec((1, H1), lambda t: (0, 0)),
            pl.BlockSpec(W2.shape, lambda t: (0, 0)),
            pl.BlockSpec((1, H2), lambda t: (0, 0)),
            pl.BlockSpec(W3.shape, lambda t: (0, 0)),
            pl.BlockSpec((1, H3), lambda t: (0, 0)),
        ],
        out_specs=pl.BlockSpec((T, H3), lambda t: (t, 0)),
        out_shape=jax.ShapeDtypeStruct((R, H3), jnp.float32),
    )(gath, q, W1x, W1f, b1.reshape(1, -1), W2, b2.reshape(1, -1), W3,
      b3.reshape(1, -1))


def _cv_body(x_ref, q_ref, f1_ref, w1x, w1m, w1f, b1, w12, b12, w13, b13,
             w21, b21, w22, b22, out_ref, *, T, k):
    x = x_ref[...]
    q = q_ref[...]
    f1 = f1_ref[...]
    qb = jnp.broadcast_to(q[:, None, :], (T, k, 3)).reshape(T * k, 3)
    xr = x[:, :3] - qb
    pre = xr @ w1x[...] + x @ w1f[...] + b1[...]
    c2 = f1 @ w1m[...]
    h1 = pre.shape[-1]
    h = jnp.maximum(
        (pre.reshape(T, k, h1) + c2[:, None, :]).reshape(T * k, h1), 0.0)
    h = jnp.maximum(h @ w12[...] + b12[...], 0.0)
    h = jnp.maximum(h @ w13[...] + b13[...], 0.0)
    s = jnp.maximum(h @ w21[...] + b21[...], 0.0)
    s = jnp.maximum(s @ w22[...] + b22[...], 0.0)
    hs = s.shape[-1]
    s3 = s.reshape(T, k, hs)
    m = jnp.max(s3, axis=1, keepdims=True)
    e = jnp.exp(s3 - m)
    w = e / jnp.sum(e, axis=1, keepdims=True)
    out_ref[...] = jnp.sum(w * h.reshape(T, k, hs), axis=1)


def _cv_pallas(gath, q, f1, layers1, layers2, k, Cp2):
    (W1, b1), (W12, b12), (W13, b13) = layers1
    (W21, b21), (W22, b22) = layers2
    R, F1 = f1.shape
    T = 64 if k == 32 else 128
    H1 = W1.shape[1]
    W1x = W1[:3]
    W1m = W1[3:3 + F1]
    F2 = W1.shape[0] - 3 - F1
    W1f = jnp.concatenate(
        [jnp.zeros((3, H1), jnp.float32), W1[3 + F1:],
         jnp.zeros((Cp2 - 3 - F2, H1), jnp.float32)], axis=0)
    grid = (R // T,)
    Hs = W22.shape[1]
    return pl.pallas_call(
        functools.partial(_cv_body, T=T, k=k),
        grid=grid,
        in_specs=[
            pl.BlockSpec((T * k, Cp2), lambda t: (t, 0)),
            pl.BlockSpec((T, 3), lambda t: (t, 0)),
            pl.BlockSpec((T, F1), lambda t: (t, 0)),
            pl.BlockSpec(W1x.shape, lambda t: (0, 0)),
            pl.BlockSpec(W1m.shape, lambda t: (0, 0)),
            pl.BlockSpec(W1f.shape, lambda t: (0, 0)),
            pl.BlockSpec((1, H1), lambda t: (0, 0)),
            pl.BlockSpec(W12.shape, lambda t: (0, 0)),
            pl.BlockSpec((1, W12.shape[1]), lambda t: (0, 0)),
            pl.BlockSpec(W13.shape, lambda t: (0, 0)),
            pl.BlockSpec((1, W13.shape[1]), lambda t: (0, 0)),
            pl.BlockSpec(W21.shape, lambda t: (0, 0)),
            pl.BlockSpec((1, W21.shape[1]), lambda t: (0, 0)),
            pl.BlockSpec(W22.shape, lambda t: (0, 0)),
            pl.BlockSpec((1, W22.shape[1]), lambda t: (0, 0)),
        ],
        out_specs=pl.BlockSpec((T, Hs), lambda t: (t, 0)),
        out_shape=jax.ShapeDtypeStruct((R, Hs), jnp.float32),
    )(gath, q, f1, W1x, W1m, W1f, b1.reshape(1, -1), W12,
      b12.reshape(1, -1), W13, b13.reshape(1, -1), W21, b21.reshape(1, -1),
      W22, b22.reshape(1, -1))


def _recover_body(cv3_ref, cv2_ref, cv1_ref, rt_ref, w1, b1, w2, b2, w3a,
                  w3b, b3, w4, b4, out_ref, *, B, M, P3, P2, P1):
    embs = []
    for b in range(B):
        m3 = jnp.mean(cv3_ref[b * P3:(b + 1) * P3], axis=0, keepdims=True)
        m2 = jnp.mean(cv2_ref[b * P2:(b + 1) * P2], axis=0, keepdims=True)
        m1 = jnp.mean(cv1_ref[b * P1:(b + 1) * P1], axis=0, keepdims=True)
        embs.append(jnp.concatenate([m3, m2, m1], axis=1))
    emb = jnp.concatenate(embs, axis=0)
    h = jnp.maximum(emb @ w1[...] + b1[...], 0.0)
    h = jnp.maximum(h @ w2[...] + b2[...], 0.0)
    hb = jnp.concatenate(
        [jnp.broadcast_to(h[b:b + 1, :], (M, h.shape[-1])) for b in range(B)],
        axis=0)
    g = jnp.maximum(hb @ w3a[...] + rt_ref[...] @ w3b[...] + b3[...], 0.0)
    out_ref[...] = g @ w4[...] + b4[...]


def _recover_pallas(cv3, cv2, cv1, rtheta, layers):
    (W1, b1), (W2, b2), (W3, b3), (W4, b4) = layers
    B, M, _ = rtheta.shape
    P3, P2, P1 = cv3.shape[0] // B, cv2.shape[0] // B, cv1.shape[0] // B
    W3a, W3b = W3[:W2.shape[1], :], W3[W2.shape[1]:, :]
    rt_flat = rtheta.reshape(B * M, 2)
    out = pl.pallas_call(
        functools.partial(_recover_body, B=B, M=M, P3=P3, P2=P2, P1=P1),
        out_shape=jax.ShapeDtypeStruct((B * M, W4.shape[1]), jnp.float32),
    )(cv3, cv2, cv1, rt_flat, W1, b1.reshape(1, -1), W2, b2.reshape(1, -1),
      W3a, W3b, b3.reshape(1, -1), W4, b4.reshape(1, -1))
    return out.reshape(B, M, W4.shape[1])


def _sa_level(xyz, feats, npoint, k, layers, use_feats):
    G, N, _ = xyz.shape
    stride = N // npoint
    new_xyz = xyz[:, ::stride, :]
    D = _dist_pallas(new_xyz, xyz)
    if use_feats:
        Cp = ((3 + feats.shape[-1]) + 15) // 16 * 16
        table = _pad_cols(
            jnp.concatenate([xyz, feats], axis=-1), Cp).reshape(G * N, Cp)
    else:
        Cp = 16
        table = _pad_cols(xyz, Cp).reshape(G * N, Cp)
    gath = _topk_gather(D.reshape(G * npoint, N), k, npoint, N, table)
    q_flat = new_xyz.reshape(G * npoint, 3)
    f = _sa_mlp_pallas(gath, q_flat, layers, k, Cp)
    return new_xyz, f


def _cv_level(xyz1, f1, xyz2, f2, k, layers1, layers2):
    B, P, _ = xyz1.shape
    D = _dist_pallas(xyz1, xyz2)
    F2 = f2.shape[-1]
    Cp2 = ((3 + F2) + 15) // 16 * 16
    table = _pad_cols(
        jnp.concatenate([xyz2.reshape(B * P, 3), f2], axis=-1), Cp2)
    gath = _topk_gather(D.reshape(B * P, P), k, P, P, table)
    return _cv_pallas(gath, xyz1.reshape(B * P, 3), f1, layers1, layers2, k,
                      Cp2)


@jax.jit
def _run(xyz_f1, features_f1, xyz_f2, features_f2, rtheta, params):
    p = params
    B = xyz_f1.shape[0]
    xyz = jnp.concatenate([xyz_f1, xyz_f2], axis=0)
    feats = jnp.concatenate([features_f1, features_f2], axis=0)

    x_1, f_1 = _sa_level(xyz, None, 1024, 32, p['psa1'], False)
    f_1r = f_1.reshape(2 * B, 1024, -1)
    x_2, f_2 = _sa_level(x_1, f_1r, 512, 16, p['psa2'], True)
    f_2r = f_2.reshape(2 * B, 512, -1)
    x_3, f_3 = _sa_level(x_2, f_2r, 256, 16, p['psa3'], True)
    f_3r = f_3.reshape(2 * B, 256, -1)

    cv3 = _cv_level(x_3[:B], f_3[:B * 256], x_3[B:], f_3[B * 256:],
                    16, p['cv3_1'], p['cv3_2'])
    cv2 = _cv_level(x_2[:B], f_2[:B * 512], x_2[B:], f_2[B * 512:],
                    16, p['cv2_1'], p['cv2_2'])
    cv1 = _cv_level(x_1[:B], f_1[:B * 1024], x_1[B:], f_1[B * 1024:],
                    32, p['cv1_1'], p['cv1_2'])

    return _recover_pallas(cv3, cv2, cv1, rtheta, p['rec'])


def kernel(xyz_f1, features_f1, xyz_f2, features_f2, rtheta, params, nout):
    return _run(xyz_f1, features_f1, xyz_f2, features_f2, rtheta, params)


# speedup vs baseline: 1.0809x; 1.0809x over previous
import functools

import jax
import jax.numpy as jnp
from jax import lax
from jax.experimental import pallas as pl
from jax.experimental.pallas import tpu as pltpu
from jax.experimental.pallas import tpu_sc as plsc

_NC = 2
_NS = 16
_NW = _NC * _NS
_L = 16


def _pad_cols(x, c):
    if x.shape[-1] == c:
        return x
    pad = [(0, 0)] * (x.ndim - 1) + [(0, c - x.shape[-1])]
    return jnp.pad(x, pad)


def _dist_body(q_ref, rt_ref, out_ref):
    q = q_ref[0]
    rt = rt_ref[0]
    dot = lax.dot_general(q, rt, (((1,), (0,)), ((), ())),
                          preferred_element_type=jnp.float32)
    qn = jnp.sum(q * q, axis=1, keepdims=True)
    rn = jnp.sum(rt * rt, axis=0, keepdims=True)
    out_ref[0] = qn + rn - 2.0 * dot


def _dist_pallas(queries, refs):
    G, P, _ = queries.shape
    N = refs.shape[1]
    refsT = jnp.swapaxes(refs, 1, 2)
    TQ = min(P, max(128, (1 << 23) // (4 * N)))
    grid = (G, P // TQ)
    return pl.pallas_call(
        _dist_body,
        grid=grid,
        in_specs=[
            pl.BlockSpec((1, TQ, 3), lambda g, t: (g, t, 0)),
            pl.BlockSpec((1, 3, N), lambda g, t: (g, 0, 0)),
        ],
        out_specs=pl.BlockSpec((1, TQ, N), lambda g, t: (g, t, 0)),
        out_shape=jax.ShapeDtypeStruct((G, P, N), jnp.float32),
    )(queries, refsT)


def _merge16(ad, ai, cd, ci):
    rd = lax.rev(cd, (0,))
    ri = lax.rev(ci, (0,))
    m = ad <= rd
    lo_d = jnp.where(m, ad, rd)
    lo_i = jnp.where(m, ai, ri)
    hi_d = jnp.where(m, rd, ad)
    hi_i = jnp.where(m, ri, ai)
    return lo_d, lo_i, hi_d, hi_i


def _topk_gather(D, k, npoints, nrefs, table):
    R, N = D.shape
    rpw = R // _NW
    nquads = rpw // 4
    CH = 8
    nchunk = N // (CH * _L)
    log2np = npoints.bit_length() - 1
    log2n = nrefs.bit_length() - 1
    nh = k // _L
    CAP = 1024
    C = table.shape[1]
    mesh = plsc.VectorSubcoreMesh(core_axis_name="c", subcore_axis_name="s")

    @functools.partial(
        pl.kernel,
        out_type=jax.ShapeDtypeStruct((R * k, C), jnp.float32),
        mesh=mesh,
        compiler_params=pltpu.CompilerParams(needs_layout_passes=False,
                                             use_tc_tiling_on_sc=False),
        scratch_types=[
            pltpu.VMEM((4, N), jnp.float32),
            pltpu.VMEM((CAP + 2 * _L,), jnp.float32),
            pltpu.VMEM((CAP + 2 * _L,), jnp.int32),
            pltpu.VMEM((k,), jnp.float32),
            pltpu.VMEM((4, k), jnp.int32),
            pltpu.VMEM((4, k, C), jnp.float32),
            [pltpu.SemaphoreType.DMA] * 4,
            [pltpu.SemaphoreType.DMA] * 4,
        ],
    )
    def kern(d_hbm, t_hbm, out_hbm, rowbuf, candd, candi, topd, topi2,
             grows, rsems, gsems):
        wid = lax.axis_index("s") * _NC + lax.axis_index("c")
        base = wid * rpw
        big = jnp.full((_L,), 3.0e38, jnp.float32)
        zero = jnp.zeros((_L,), jnp.int32)
        ones = jnp.full((_L,), 1, jnp.int32)
        capv = jnp.full((_L,), CAP - 1, jnp.int32)

        def merge_cand(topi, cd0, ci0):
            cd, ci = plsc.sort_key_val(cd0, ci0)
            if nh == 1:
                ad = topd[pl.ds(0, _L)]
                ai = topi[pl.ds(0, _L)]
                lo_d, lo_i, _hd, _hi = _merge16(ad, ai, cd, ci)
                nd, ni = plsc.sort_key_val(lo_d, lo_i)
                topd[pl.ds(0, _L)] = nd
                topi[pl.ds(0, _L)] = ni
            else:
                a0d = topd[pl.ds(0, _L)]
                a0i = topi[pl.ds(0, _L)]
                a1d = topd[pl.ds(_L, _L)]
                a1i = topi[pl.ds(_L, _L)]
                ld, li, _hd, _hi = _merge16(a1d, a1i, cd, ci)
                sd, si = plsc.sort_key_val(ld, li)
                l0d, l0i, h0d, h0i = _merge16(a0d, a0i, sd, si)
                n0d, n0i = plsc.sort_key_val(l0d, l0i)
                n1d, n1i = plsc.sort_key_val(h0d, h0i)
                topd[pl.ds(0, _L)] = n0d
                topi[pl.ds(0, _L)] = n0i
                topd[pl.ds(_L, _L)] = n1d
                topi[pl.ds(_L, _L)] = n1i

        def process(phase, row):
            topi = topi2.at[phase]
            for h in range(nh):
                topd[pl.ds(h * _L, _L)] = big
                topi[pl.ds(h * _L, _L)] = zero

            def folda(c, carry):
                a1, a2 = carry
                b = c * (CH * _L)
                for t in range(CH):
                    v = rowbuf[phase, pl.ds(b + t * _L, _L)]
                    a2 = jnp.minimum(a2, jnp.maximum(a1, v))
                    a1 = jnp.minimum(a1, v)
                return a1, a2

            a1, a2 = lax.fori_loop(0, nchunk, folda, (big, big))
            s2, _si = plsc.sort_key_val(a2, zero)
            tvec = jnp.broadcast_to(s2[_L - 1], (_L,))

            def collect(c, curv):
                b = c * (CH * _L)
                ss = []
                for t in range(CH):
                    v = rowbuf[phase, pl.ds(b + t * _L, _L)]
                    m = v <= tvec
                    ss.append((v, m, plsc.cumsum(jnp.where(m, ones, zero))))
                cur = curv
                for t in range(CH):
                    v, m, s = ss[t]
                    idx = jnp.minimum(cur + s - 1, capv)
                    vi = b + t * _L + lax.iota(jnp.int32, _L)
                    plsc.store_scatter(candd, [idx], v, mask=m)
                    plsc.store_scatter(candi, [idx], vi, mask=m)
                    cur = cur + jnp.broadcast_to(s[_L - 1], (_L,))
                return cur

            curv = lax.fori_loop(0, nchunk, collect, zero)

            mtot = jnp.minimum(jnp.minimum(curv, capv)[0], jnp.int32(CAP))
            candd[pl.ds(mtot, _L)] = big
            nb = lax.shift_right_logical(mtot + (_L - 1), 4)

            def mergeb(i, _):
                b = i * _L
                merge_cand(topi, candd[pl.ds(b, _L)], candi[pl.ds(b, _L)])
                return 0

            lax.fori_loop(0, nb, mergeb, 0)

            off = lax.shift_left(lax.shift_right_logical(row, log2np), log2n)
            for h in range(nh):
                topi[pl.ds(h * _L, _L)] = topi[pl.ds(h * _L, _L)] + off

        def drain_gather(dph, drow):
            pltpu.make_async_copy(t_hbm.at[topi2.at[dph]], grows.at[dph],
                                  gsems[dph]).wait()
            pltpu.sync_copy(grows.at[dph], out_hbm.at[pl.ds(drow * k, k)])

        pltpu.async_copy(d_hbm.at[base], rowbuf.at[0], rsems[0])
        pltpu.async_copy(d_hbm.at[base + 1], rowbuf.at[1], rsems[1])

        def quad(q, _):
            rowq = base + 4 * q
            for ph in range(4):
                row = rowq + ph
                pltpu.make_async_copy(d_hbm.at[row], rowbuf.at[ph],
                                      rsems[ph]).wait()
                pph = (ph + 2) % 4

                @pl.when(row + 2 < base + rpw)
                def _():
                    pltpu.async_copy(d_hbm.at[row + 2], rowbuf.at[pph],
                                     rsems[pph])

                process(ph, row)

                @pl.when(row - 2 >= base)
                def _():
                    drain_gather(pph, row - 2)

                pltpu.async_copy(t_hbm.at[topi2.at[ph]], grows.at[ph],
                                 gsems[ph])
            return 0

        lax.fori_loop(0, nquads, quad, 0)
        drain_gather(2, base + rpw - 2)
        drain_gather(3, base + rpw - 1)

    return kern(D, table)


def _sa_body(x_ref, q_ref, w1x, w1f, b1, w2, b2, w3, b3, out_ref, *, T, k):
    x = x_ref[...]
    q = q_ref[...]
    h1 = w1x.shape[1]
    qb = jnp.broadcast_to(q[:, None, :], (T, k, 3)).reshape(T * k, 3)
    xr = x[:, :3] - qb
    h = jnp.maximum(xr @ w1x[...] + x @ w1f[...] + b1[...], 0.0)
    h = jnp.maximum(h @ w2[...] + b2[...], 0.0)
    h = jnp.maximum(h @ w3[...] + b3[...], 0.0)
    out_ref[...] = jnp.max(h.reshape(T, k, h.shape[-1]), axis=1)


def _sa_mlp_pallas(gath, q, layers, k, Cp):
    (W1, b1), (W2, b2), (W3, b3) = layers
    R = q.shape[0]
    T = 64 if k == 32 else 128
    H1, H2, H3 = W1.shape[1], W2.shape[1], W3.shape[1]
    W1x = W1[:3]
    W1f = _pad_cols(W1[3:].T, 0).T if False else jnp.concatenate(
        [jnp.zeros((3, H1), jnp.float32), W1[3:],
         jnp.zeros((Cp - W1.shape[0], H1), jnp.float32)], axis=0)
    grid = (R // T,)
    return pl.pallas_call(
        functools.partial(_sa_body, T=T, k=k),
        grid=grid,
        in_specs=[
            pl.BlockSpec((T * k, Cp), lambda t: (t, 0)),
            pl.BlockSpec((T, 3), lambda t: (t, 0)),
            pl.BlockSpec(W1x.shape, lambda t: (0, 0)),
            pl.BlockSpec(W1f.shape, lambda t: (0, 0)),
            pl.BlockSpec((1, H1), lambda t: (0, 0)),
            pl.BlockSpec(W2.shape, lambda t: (0, 0)),
            pl.BlockSpec((1, H2), lambda t: (0, 0)),
            pl.BlockSpec(W3.shape, lambda t: (0, 0)),
            pl.BlockSpec((1, H3), lambda t: (0, 0)),
        ],
        out_specs=pl.BlockSpec((T, H3), lambda t: (t, 0)),
        out_shape=jax.ShapeDtypeStruct((R, H3), jnp.float32),
    )(gath, q, W1x, W1f, b1.reshape(1, -1), W2, b2.reshape(1, -1), W3,
      b3.reshape(1, -1))


def _cv_body(x_ref, q_ref, f1_ref, w1x, w1m, w1f, b1, w12, b12, w13, b13,
             w21, b21, w22, b22, out_ref, *, T, k):
    x = x_ref[...]
    q = q_ref[...]
    f1 = f1_ref[...]
    qb = jnp.broadcast_to(q[:, None, :], (T, k, 3)).reshape(T * k, 3)
    xr = x[:, :3] - qb
    pre = xr @ w1x[...] + x @ w1f[...] + b1[...]
    c2 = f1 @ w1m[...]
    h1 = pre.shape[-1]
    h = jnp.maximum(
        (pre.reshape(T, k, h1) + c2[:, None, :]).reshape(T * k, h1), 0.0)
    h = jnp.maximum(h @ w12[...] + b12[...], 0.0)
    h = jnp.maximum(h @ w13[...] + b13[...], 0.0)
    s = jnp.maximum(h @ w21[...] + b21[...], 0.0)
    s = jnp.maximum(s @ w22[...] + b22[...], 0.0)
    hs = s.shape[-1]
    s3 = s.reshape(T, k, hs)
    m = jnp.max(s3, axis=1, keepdims=True)
    e = jnp.exp(s3 - m)
    w = e / jnp.sum(e, axis=1, keepdims=True)
    out_ref[...] = jnp.sum(w * h.reshape(T, k, hs), axis=1)


def _cv_pallas(gath, q, f1, layers1, layers2, k, Cp2):
    (W1, b1), (W12, b12), (W13, b13) = layers1
    (W21, b21), (W22, b22) = layers2
    R, F1 = f1.shape
    T = 64 if k == 32 else 128
    H1 = W1.shape[1]
    W1x = W1[:3]
    W1m = W1[3:3 + F1]
    F2 = W1.shape[0] - 3 - F1
    W1f = jnp.concatenate(
        [jnp.zeros((3, H1), jnp.float32), W1[3 + F1:],
         jnp.zeros((Cp2 - 3 - F2, H1), jnp.float32)], axis=0)
    grid = (R // T,)
    Hs = W22.shape[1]
    return pl.pallas_call(
        functools.partial(_cv_body, T=T, k=k),
        grid=grid,
        in_specs=[
            pl.BlockSpec((T * k, Cp2), lambda t: (t, 0)),
            pl.BlockSpec((T, 3), lambda t: (t, 0)),
            pl.BlockSpec((T, F1), lambda t: (t, 0)),
            pl.BlockSpec(W1x.shape, lambda t: (0, 0)),
            pl.BlockSpec(W1m.shape, lambda t: (0, 0)),
            pl.BlockSpec(W1f.shape, lambda t: (0, 0)),
            pl.BlockSpec((1, H1), lambda t: (0, 0)),
            pl.BlockSpec(W12.shape, lambda t: (0, 0)),
            pl.BlockSpec((1, W12.shape[1]), lambda t: (0, 0)),
            pl.BlockSpec(W13.shape, lambda t: (0, 0)),
            pl.BlockSpec((1, W13.shape[1]), lambda t: (0, 0)),
            pl.BlockSpec(W21.shape, lambda t: (0, 0)),
            pl.BlockSpec((1, W21.shape[1]), lambda t: (0, 0)),
            pl.BlockSpec(W22.shape, lambda t: (0, 0)),
            pl.BlockSpec((1, W22.shape[1]), lambda t: (0, 0)),
        ],
        out_specs=pl.BlockSpec((T, Hs), lambda t: (t, 0)),
        out_shape=jax.ShapeDtypeStruct((R, Hs), jnp.float32),
    )(gath, q, f1, W1x, W1m, W1f, b1.reshape(1, -1), W12,
      b12.reshape(1, -1), W13, b13.reshape(1, -1), W21, b21.reshape(1, -1),
      W22, b22.reshape(1, -1))


def _recover_body(cv3_ref, cv2_ref, cv1_ref, rt_ref, w1, b1, w2, b2, w3a,
                  w3b, b3, w4, b4, out_ref, *, B, M, P3, P2, P1):
    embs = []
    for b in range(B):
        m3 = jnp.mean(cv3_ref[b * P3:(b + 1) * P3], axis=0, keepdims=True)
        m2 = jnp.mean(cv2_ref[b * P2:(b + 1) * P2], axis=0, keepdims=True)
        m1 = jnp.mean(cv1_ref[b * P1:(b + 1) * P1], axis=0, keepdims=True)
        embs.append(jnp.concatenate([m3, m2, m1], axis=1))
    emb = jnp.concatenate(embs, axis=0)
    h = jnp.maximum(emb @ w1[...] + b1[...], 0.0)
    h = jnp.maximum(h @ w2[...] + b2[...], 0.0)
    hb = jnp.concatenate(
        [jnp.broadcast_to(h[b:b + 1, :], (M, h.shape[-1])) for b in range(B)],
        axis=0)
    g = jnp.maximum(hb @ w3a[...] + rt_ref[...] @ w3b[...] + b3[...], 0.0)
    out_ref[...] = g @ w4[...] + b4[...]


def _recover_pallas(cv3, cv2, cv1, rtheta, layers):
    (W1, b1), (W2, b2), (W3, b3), (W4, b4) = layers
    B, M, _ = rtheta.shape
    P3, P2, P1 = cv3.shape[0] // B, cv2.shape[0] // B, cv1.shape[0] // B
    W3a, W3b = W3[:W2.shape[1], :], W3[W2.shape[1]:, :]
    rt_flat = rtheta.reshape(B * M, 2)
    out = pl.pallas_call(
        functools.partial(_recover_body, B=B, M=M, P3=P3, P2=P2, P1=P1),
        out_shape=jax.ShapeDtypeStruct((B * M, W4.shape[1]), jnp.float32),
    )(cv3, cv2, cv1, rt_flat, W1, b1.reshape(1, -1), W2, b2.reshape(1, -1),
      W3a, W3b, b3.reshape(1, -1), W4, b4.reshape(1, -1))
    return out.reshape(B, M, W4.shape[1])


def _sa_level(xyz, feats, npoint, k, layers, use_feats):
    G, N, _ = xyz.shape
    stride = N // npoint
    new_xyz = xyz[:, ::stride, :]
    D = _dist_pallas(new_xyz, xyz)
    if use_feats:
        Cp = ((3 + feats.shape[-1]) + 15) // 16 * 16
        table = _pad_cols(
            jnp.concatenate([xyz, feats], axis=-1), Cp).reshape(G * N, Cp)
    else:
        Cp = 16
        table = _pad_cols(xyz, Cp).reshape(G * N, Cp)
    gath = _topk_gather(D.reshape(G * npoint, N), k, npoint, N, table)
    q_flat = new_xyz.reshape(G * npoint, 3)
    f = _sa_mlp_pallas(gath, q_flat, layers, k, Cp)
    return new_xyz, f


def _cv_level(xyz1, f1, xyz2, f2, k, layers1, layers2):
    B, P, _ = xyz1.shape
    D = _dist_pallas(xyz1, xyz2)
    F2 = f2.shape[-1]
    Cp2 = ((3 + F2) + 15) // 16 * 16
    table = _pad_cols(
        jnp.concatenate([xyz2.reshape(B * P, 3), f2], axis=-1), Cp2)
    gath = _topk_gather(D.reshape(B * P, P), k, P, P, table)
    return _cv_pallas(gath, xyz1.reshape(B * P, 3), f1, layers1, layers2, k,
                      Cp2)


@jax.jit
def _run(xyz_f1, features_f1, xyz_f2, features_f2, rtheta, params):
    p = params
    B = xyz_f1.shape[0]
    xyz = jnp.concatenate([xyz_f1, xyz_f2], axis=0)
    feats = jnp.concatenate([features_f1, features_f2], axis=0)

    x_1, f_1 = _sa_level(xyz, None, 1024, 32, p['psa1'], False)
    f_1r = f_1.reshape(2 * B, 1024, -1)
    x_2, f_2 = _sa_level(x_1, f_1r, 512, 16, p['psa2'], True)
    f_2r = f_2.reshape(2 * B, 512, -1)
    x_3, f_3 = _sa_level(x_2, f_2r, 256, 16, p['psa3'], True)
    f_3r = f_3.reshape(2 * B, 256, -1)

    cv3 = _cv_level(x_3[:B], f_3[:B * 256], x_3[B:], f_3[B * 256:],
                    16, p['cv3_1'], p['cv3_2'])
    cv2 = _cv_level(x_2[:B], f_2[:B * 512], x_2[B:], f_2[B * 512:],
                    16, p['cv2_1'], p['cv2_2'])
    cv1 = _cv_level(x_1[:B], f_1[:B * 1024], x_1[B:], f_1[B * 1024:],
                    32, p['cv1_1'], p['cv1_2'])

    return _recover_pallas(cv3, cv2, cv1, rtheta, p['rec'])


def kernel(xyz_f1, features_f1, xyz_f2, features_f2, rtheta, params, nout):
    return _run(xyz_f1, features_f1, xyz_f2, features_f2, rtheta, params)


# speedup vs baseline: 1.1151x; 1.0317x over previous
import functools

import jax
import jax.numpy as jnp
from jax import lax
from jax.experimental import pallas as pl
from jax.experimental.pallas import tpu as pltpu
from jax.experimental.pallas import tpu_sc as plsc

_NC = 2
_NS = 16
_NW = _NC * _NS
_L = 16


def _pad_cols(x, c):
    if x.shape[-1] == c:
        return x
    pad = [(0, 0)] * (x.ndim - 1) + [(0, c - x.shape[-1])]
    return jnp.pad(x, pad)


def _merge16(ad, ai, cd, ci):
    rd = lax.rev(cd, (0,))
    ri = lax.rev(ci, (0,))
    m = ad <= rd
    lo_d = jnp.where(m, ad, rd)
    lo_i = jnp.where(m, ai, ri)
    hi_d = jnp.where(m, rd, ad)
    hi_i = jnp.where(m, ri, ai)
    return lo_d, lo_i, hi_d, hi_i


def _knn_gather(queries, refs, k, npoints, nrefs, table):
    R = queries.shape[0]
    G = refs.shape[0]
    N = nrefs
    rpw = R // _NW
    nquads = rpw // 4
    CH = 8
    nchunk = N // (CH * _L)
    log2np = npoints.bit_length() - 1
    nh = k // _L
    CAP = 1024
    C = table.shape[1]
    qpad = _pad_cols(queries, _L).reshape(R * _L)
    refsP = jnp.pad(jnp.swapaxes(refs, 1, 2),
                    ((0, 0), (0, 1), (0, 0))).reshape(G, 4 * N)
    mesh = plsc.VectorSubcoreMesh(core_axis_name="c", subcore_axis_name="s")

    @functools.partial(
        pl.kernel,
        out_type=jax.ShapeDtypeStruct((R * k, C), jnp.float32),
        mesh=mesh,
        compiler_params=pltpu.CompilerParams(needs_layout_passes=False,
                                             use_tc_tiling_on_sc=False),
        scratch_types=[
            pltpu.VMEM((rpw * _L,), jnp.float32),
            pltpu.VMEM((4 * N,), jnp.float32),
            pltpu.VMEM((N,), jnp.float32),
            pltpu.VMEM((CAP + 2 * _L,), jnp.float32),
            pltpu.VMEM((CAP + 2 * _L,), jnp.int32),
            pltpu.VMEM((k,), jnp.float32),
            pltpu.VMEM((4, k), jnp.int32),
            pltpu.VMEM((4, k, C), jnp.float32),
            [pltpu.SemaphoreType.DMA] * 4,
        ],
    )
    def kern(q_hbm, refs_hbm, t_hbm, out_hbm, qbuf, refsbuf, rowbuf, candd,
             candi, topd, topi2, grows, gsems):
        wid = lax.axis_index("s") * _NC + lax.axis_index("c")
        base = wid * rpw
        grp = lax.shift_right_logical(base, log2np)
        big = jnp.full((_L,), 3.0e38, jnp.float32)
        zero = jnp.zeros((_L,), jnp.int32)
        ones = jnp.full((_L,), 1, jnp.int32)
        capv = jnp.full((_L,), CAP - 1, jnp.int32)
        pltpu.sync_copy(refs_hbm.at[grp], refsbuf)
        pltpu.sync_copy(q_hbm.at[pl.ds(base * _L, rpw * _L)], qbuf)

        def merge_cand(topi, cd0, ci0):
            cd, ci = plsc.sort_key_val(cd0, ci0)
            if nh == 1:
                ad = topd[pl.ds(0, _L)]
                ai = topi[pl.ds(0, _L)]
                lo_d, lo_i, _hd, _hi = _merge16(ad, ai, cd, ci)
                nd, ni = plsc.sort_key_val(lo_d, lo_i)
                topd[pl.ds(0, _L)] = nd
                topi[pl.ds(0, _L)] = ni
            else:
                a0d = topd[pl.ds(0, _L)]
                a0i = topi[pl.ds(0, _L)]
                a1d = topd[pl.ds(_L, _L)]
                a1i = topi[pl.ds(_L, _L)]
                ld, li, _hd, _hi = _merge16(a1d, a1i, cd, ci)
                sd, si = plsc.sort_key_val(ld, li)
                l0d, l0i, h0d, h0i = _merge16(a0d, a0i, sd, si)
                n0d, n0i = plsc.sort_key_val(l0d, l0i)
                n1d, n1i = plsc.sort_key_val(h0d, h0i)
                topd[pl.ds(0, _L)] = n0d
                topi[pl.ds(0, _L)] = n0i
                topd[pl.ds(_L, _L)] = n1d
                topi[pl.ds(_L, _L)] = n1i

        off = lax.shift_left(grp, nrefs.bit_length() - 1)

        def process(phase, row):
            topi = topi2.at[phase]
            for h in range(nh):
                topd[pl.ds(h * _L, _L)] = big
                topi[pl.ds(h * _L, _L)] = zero

            rl16 = (row - base) * _L
            qv = qbuf[pl.ds(rl16, _L)]
            qx = jnp.broadcast_to(qv[0], (_L,))
            qy = jnp.broadcast_to(qv[1], (_L,))
            qz = jnp.broadcast_to(qv[2], (_L,))

            def folda(c, carry):
                a1, a2 = carry
                b = c * (CH * _L)
                for t in range(CH):
                    o = b + t * _L
                    dx = refsbuf[pl.ds(o, _L)] - qx
                    dy = refsbuf[pl.ds(N + o, _L)] - qy
                    dz = refsbuf[pl.ds(2 * N + o, _L)] - qz
                    v = dx * dx + dy * dy + dz * dz
                    rowbuf[pl.ds(o, _L)] = v
                    a2 = jnp.minimum(a2, jnp.maximum(a1, v))
                    a1 = jnp.minimum(a1, v)
                return a1, a2

            a1, a2 = lax.fori_loop(0, nchunk, folda, (big, big))
            s2, _si = plsc.sort_key_val(a2, zero)
            tvec = jnp.broadcast_to(s2[_L - 1], (_L,))

            def collect(c, curv):
                b = c * (CH * _L)
                ss = []
                for t in range(CH):
                    v = rowbuf[pl.ds(b + t * _L, _L)]
                    m = v <= tvec
                    ss.append((v, m, plsc.cumsum(jnp.where(m, ones, zero))))
                cur = curv
                for t in range(CH):
                    v, m, s = ss[t]
                    idx = jnp.minimum(cur + s - 1, capv)
                    vi = b + t * _L + lax.iota(jnp.int32, _L)
                    plsc.store_scatter(candd, [idx], v, mask=m)
                    plsc.store_scatter(candi, [idx], vi, mask=m)
                    cur = cur + jnp.broadcast_to(s[_L - 1], (_L,))
                return cur

            curv = lax.fori_loop(0, nchunk, collect, zero)

            mtot = jnp.minimum(jnp.minimum(curv, capv)[0], jnp.int32(CAP))
            candd[pl.ds(mtot, _L)] = big
            nb = lax.shift_right_logical(mtot + (_L - 1), 4)

            def mergeb(i, _):
                b = i * _L
                merge_cand(topi, candd[pl.ds(b, _L)], candi[pl.ds(b, _L)])
                return 0

            lax.fori_loop(0, nb, mergeb, 0)
            for h in range(nh):
                topi[pl.ds(h * _L, _L)] = topi[pl.ds(h * _L, _L)] + off

        def drain_gather(dph, drow):
            pltpu.make_async_copy(t_hbm.at[topi2.at[dph]], grows.at[dph],
                                  gsems[dph]).wait()
            pltpu.sync_copy(grows.at[dph], out_hbm.at[pl.ds(drow * k, k)])

        def quad(q, _):
            rowq = base + 4 * q
            for ph in range(4):
                row = rowq + ph
                pph = (ph + 2) % 4
                process(ph, row)

                @pl.when(row - 2 >= base)
                def _():
                    drain_gather(pph, row - 2)

                pltpu.async_copy(t_hbm.at[topi2.at[ph]], grows.at[ph],
                                 gsems[ph])
            return 0

        lax.fori_loop(0, nquads, quad, 0)
        drain_gather(2, base + rpw - 2)
        drain_gather(3, base + rpw - 1)

    return kern(qpad, refsP, table)


def _sa_body(x_ref, q_ref, w1x, w1f, b1, w2, b2, w3, b3, out_ref, *, T, k):
    x = x_ref[...]
    q = q_ref[...]
    h1 = w1x.shape[1]
    qb = jnp.broadcast_to(q[:, None, :], (T, k, 3)).reshape(T * k, 3)
    xr = x[:, :3] - qb
    h = jnp.maximum(xr @ w1x[...] + x @ w1f[...] + b1[...], 0.0)
    h = jnp.maximum(h @ w2[...] + b2[...], 0.0)
    h = jnp.maximum(h @ w3[...] + b3[...], 0.0)
    out_ref[...] = jnp.max(h.reshape(T, k, h.shape[-1]), axis=1)


def _sa_mlp_pallas(gath, q, layers, k, Cp):
    (W1, b1), (W2, b2), (W3, b3) = layers
    R = q.shape[0]
    T = 64 if k == 32 else 128
    H1, H2, H3 = W1.shape[1], W2.shape[1], W3.shape[1]
    W1x = W1[:3]
    W1f = _pad_cols(W1[3:].T, 0).T if False else jnp.concatenate(
        [jnp.zeros((3, H1), jnp.float32), W1[3:],
         jnp.zeros((Cp - W1.shape[0], H1), jnp.float32)], axis=0)
    grid = (R // T,)
    return pl.pallas_call(
        functools.partial(_sa_body, T=T, k=k),
        grid=grid,
        in_specs=[
            pl.BlockSpec((T * k, Cp), lambda t: (t, 0)),
            pl.BlockSpec((T, 3), lambda t: (t, 0)),
            pl.BlockSpec(W1x.shape, lambda t: (0, 0)),
            pl.BlockSpec(W1f.shape, lambda t: (0, 0)),
            pl.BlockSpec((1, H1), lambda t: (0, 0)),
            pl.BlockSpec(W2.shape, lambda t: (0, 0)),
            pl.BlockSpec((1, H2), lambda t: (0, 0)),
            pl.BlockSpec(W3.shape, lambda t: (0, 0)),
            pl.BlockSpec((1, H3), lambda t: (0, 0)),
        ],
        out_specs=pl.BlockSpec((T, H3), lambda t: (t, 0)),
        out_shape=jax.ShapeDtypeStruct((R, H3), jnp.float32),
    )(gath, q, W1x, W1f, b1.reshape(1, -1), W2, b2.reshape(1, -1), W3,
      b3.reshape(1, -1))


def _cv_body(x_ref, q_ref, f1_ref, w1x, w1m, w1f, b1, w12, b12, w13, b13,
             w21, b21, w22, b22, out_ref, *, T, k):
    x = x_ref[...]
    q = q_ref[...]
    f1 = f1_ref[...]
    qb = jnp.broadcast_to(q[:, None, :], (T, k, 3)).reshape(T * k, 3)
    xr = x[:, :3] - qb
    pre = xr @ w1x[...] + x @ w1f[...] + b1[...]
    c2 = f1 @ w1m[...]
    h1 = pre.shape[-1]
    h = jnp.maximum(
        (pre.reshape(T, k, h1) + c2[:, None, :]).reshape(T * k, h1), 0.0)
    h = jnp.maximum(h @ w12[...] + b12[...], 0.0)
    h = jnp.maximum(h @ w13[...] + b13[...], 0.0)
    s = jnp.maximum(h @ w21[...] + b21[...], 0.0)
    s = jnp.maximum(s @ w22[...] + b22[...], 0.0)
    hs = s.shape[-1]
    s3 = s.reshape(T, k, hs)
    m = jnp.max(s3, axis=1, keepdims=True)
    e = jnp.exp(s3 - m)
    w = e / jnp.sum(e, axis=1, keepdims=True)
    out_ref[...] = jnp.sum(w * h.reshape(T, k, hs), axis=1)


def _cv_pallas(gath, q, f1, layers1, layers2, k, Cp2):
    (W1, b1), (W12, b12), (W13, b13) = layers1
    (W21, b21), (W22, b22) = layers2
    R, F1 = f1.shape
    T = 64 if k == 32 else 128
    H1 = W1.shape[1]
    W1x = W1[:3]
    W1m = W1[3:3 + F1]
    F2 = W1.shape[0] - 3 - F1
    W1f = jnp.concatenate(
        [jnp.zeros((3, H1), jnp.float32), W1[3 + F1:],
         jnp.zeros((Cp2 - 3 - F2, H1), jnp.float32)], axis=0)
    grid = (R // T,)
    Hs = W22.shape[1]
    return pl.pallas_call(
        functools.partial(_cv_body, T=T, k=k),
        grid=grid,
        in_specs=[
            pl.BlockSpec((T * k, Cp2), lambda t: (t, 0)),
            pl.BlockSpec((T, 3), lambda t: (t, 0)),
            pl.BlockSpec((T, F1), lambda t: (t, 0)),
            pl.BlockSpec(W1x.shape, lambda t: (0, 0)),
            pl.BlockSpec(W1m.shape, lambda t: (0, 0)),
            pl.BlockSpec(W1f.shape, lambda t: (0, 0)),
            pl.BlockSpec((1, H1), lambda t: (0, 0)),
            pl.BlockSpec(W12.shape, lambda t: (0, 0)),
            pl.BlockSpec((1, W12.shape[1]), lambda t: (0, 0)),
            pl.BlockSpec(W13.shape, lambda t: (0, 0)),
            pl.BlockSpec((1, W13.shape[1]), lambda t: (0, 0)),
            pl.BlockSpec(W21.shape, lambda t: (0, 0)),
            pl.BlockSpec((1, W21.shape[1]), lambda t: (0, 0)),
            pl.BlockSpec(W22.shape, lambda t: (0, 0)),
            pl.BlockSpec((1, W22.shape[1]), lambda t: (0, 0)),
        ],
        out_specs=pl.BlockSpec((T, Hs), lambda t: (t, 0)),
        out_shape=jax.ShapeDtypeStruct((R, Hs), jnp.float32),
    )(gath, q, f1, W1x, W1m, W1f, b1.reshape(1, -1), W12,
      b12.reshape(1, -1), W13, b13.reshape(1, -1), W21, b21.reshape(1, -1),
      W22, b22.reshape(1, -1))


def _recover_body(cv3_ref, cv2_ref, cv1_ref, rt_ref, w1, b1, w2, b2, w3a,
                  w3b, b3, w4, b4, out_ref, *, B, M, P3, P2, P1):
    embs = []
    for b in range(B):
        m3 = jnp.mean(cv3_ref[b * P3:(b + 1) * P3], axis=0, keepdims=True)
        m2 = jnp.mean(cv2_ref[b * P2:(b + 1) * P2], axis=0, keepdims=True)
        m1 = jnp.mean(cv1_ref[b * P1:(b + 1) * P1], axis=0, keepdims=True)
        embs.append(jnp.concatenate([m3, m2, m1], axis=1))
    emb = jnp.concatenate(embs, axis=0)
    h = jnp.maximum(emb @ w1[...] + b1[...], 0.0)
    h = jnp.maximum(h @ w2[...] + b2[...], 0.0)
    hb = jnp.concatenate(
        [jnp.broadcast_to(h[b:b + 1, :], (M, h.shape[-1])) for b in range(B)],
        axis=0)
    g = jnp.maximum(hb @ w3a[...] + rt_ref[...] @ w3b[...] + b3[...], 0.0)
    out_ref[...] = g @ w4[...] + b4[...]


def _recover_pallas(cv3, cv2, cv1, rtheta, layers):
    (W1, b1), (W2, b2), (W3, b3), (W4, b4) = layers
    B, M, _ = rtheta.shape
    P3, P2, P1 = cv3.shape[0] // B, cv2.shape[0] // B, cv1.shape[0] // B
    W3a, W3b = W3[:W2.shape[1], :], W3[W2.shape[1]:, :]
    rt_flat = rtheta.reshape(B * M, 2)
    out = pl.pallas_call(
        functools.partial(_recover_body, B=B, M=M, P3=P3, P2=P2, P1=P1),
        out_shape=jax.ShapeDtypeStruct((B * M, W4.shape[1]), jnp.float32),
    )(cv3, cv2, cv1, rt_flat, W1, b1.reshape(1, -1), W2, b2.reshape(1, -1),
      W3a, W3b, b3.reshape(1, -1), W4, b4.reshape(1, -1))
    return out.reshape(B, M, W4.shape[1])


def _sa_level(xyz, feats, npoint, k, layers, use_feats):
    G, N, _ = xyz.shape
    stride = N // npoint
    new_xyz = xyz[:, ::stride, :]
    if use_feats:
        Cp = ((3 + feats.shape[-1]) + 15) // 16 * 16
        table = _pad_cols(
            jnp.concatenate([xyz, feats], axis=-1), Cp).reshape(G * N, Cp)
    else:
        Cp = 16
        table = _pad_cols(xyz, Cp).reshape(G * N, Cp)
    q_flat = new_xyz.reshape(G * npoint, 3)
    gath = _knn_gather(q_flat, xyz, k, npoint, N, table)
    f = _sa_mlp_pallas(gath, q_flat, layers, k, Cp)
    return new_xyz, f


def _cv_level(xyz1, f1, xyz2, f2, k, layers1, layers2):
    B, P, _ = xyz1.shape
    F2 = f2.shape[-1]
    Cp2 = ((3 + F2) + 15) // 16 * 16
    table = _pad_cols(
        jnp.concatenate([xyz2.reshape(B * P, 3), f2], axis=-1), Cp2)
    gath = _knn_gather(xyz1.reshape(B * P, 3), xyz2, k, P, P, table)
    return _cv_pallas(gath, xyz1.reshape(B * P, 3), f1, layers1, layers2, k,
                      Cp2)


@jax.jit
def _run(xyz_f1, features_f1, xyz_f2, features_f2, rtheta, params):
    p = params
    B = xyz_f1.shape[0]
    xyz = jnp.concatenate([xyz_f1, xyz_f2], axis=0)
    feats = jnp.concatenate([features_f1, features_f2], axis=0)

    x_1, f_1 = _sa_level(xyz, None, 1024, 32, p['psa1'], False)
    f_1r = f_1.reshape(2 * B, 1024, -1)
    x_2, f_2 = _sa_level(x_1, f_1r, 512, 16, p['psa2'], True)
    f_2r = f_2.reshape(2 * B, 512, -1)
    x_3, f_3 = _sa_level(x_2, f_2r, 256, 16, p['psa3'], True)
    f_3r = f_3.reshape(2 * B, 256, -1)

    cv3 = _cv_level(x_3[:B], f_3[:B * 256], x_3[B:], f_3[B * 256:],
                    16, p['cv3_1'], p['cv3_2'])
    cv2 = _cv_level(x_2[:B], f_2[:B * 512], x_2[B:], f_2[B * 512:],
                    16, p['cv2_1'], p['cv2_2'])
    cv1 = _cv_level(x_1[:B], f_1[:B * 1024], x_1[B:], f_1[B * 1024:],
                    32, p['cv1_1'], p['cv1_2'])

    return _recover_pallas(cv3, cv2, cv1, rtheta, p['rec'])


def kernel(xyz_f1, features_f1, xyz_f2, features_f2, rtheta, params, nout):
    return _run(xyz_f1, features_f1, xyz_f2, features_f2, rtheta, params)
